# baseline (speedup 1.0000x reference)
import functools

import jax
import jax.numpy as jnp
from jax import lax
from jax.experimental import pallas as pl
from jax.experimental.pallas import tpu as pltpu
from jax.experimental.pallas import tpu_sc as plsc

_B = 4096
_L = 200
_EMB = 64
_OUT = 128

_NW = 32
_CH = 128
_NBUF = 5
_NPIPE = 4
_BP = _B // _NPIPE
_CHUNKS = (_BP * _L) // (_NW * _CH)
_NGROUP = _CHUNKS // _NBUF

_BB = 64
_LH = _L // 2
_SENT = -1e35


def _sc_gather(tok2, table):
    mesh = plsc.VectorSubcoreMesh(core_axis_name="c", subcore_axis_name="s")

    @functools.partial(
        pl.kernel,
        mesh=mesh,
        out_type=jax.ShapeDtypeStruct((_BP * _L, _EMB), jnp.float32),
        scratch_types=(
            [pltpu.VMEM((_CHUNKS, _CH), jnp.int32),
             pltpu.VMEM((_NBUF, _CH, _EMB), jnp.float32)]
            + [pltpu.SemaphoreType.DMA] * _NBUF
        ),
        compiler_params=pltpu.CompilerParams(use_tc_tiling_on_sc=False),
    )
    def k(tok_hbm, table_hbm, out_hbm, idx_v, rows_v, *sems):
        wid = lax.axis_index("s") * 2 + lax.axis_index("c")
        pltpu.sync_copy(tok_hbm.at[pl.ds(wid * _CHUNKS, _CHUNKS)], idx_v)
        for t in range(_NBUF):
            pltpu.async_copy(table_hbm.at[idx_v.at[t]], rows_v.at[t], sems[t])

        def body(g, carry):
            for t in range(_NBUF):
                j = g * _NBUF + t
                pltpu.make_async_copy(
                    table_hbm.at[idx_v.at[j]], rows_v.at[t], sems[t]
                ).wait()
                pltpu.sync_copy(
                    rows_v.at[t],
                    out_hbm.at[pl.ds((wid * _CHUNKS + j) * _CH, _CH)],
                )

                @pl.when(g < _NGROUP - 1)
                def _():
                    pltpu.async_copy(
                        table_hbm.at[idx_v.at[j + _NBUF]], rows_v.at[t], sems[t]
                    )

            return carry

        lax.fori_loop(0, _NGROUP, body, 0)

    return k(tok2, table)


def _tc_body(x_ref, wt_ref, b_ref, o_ref):
    x0 = x_ref[...]
    sent = jnp.float32(_SENT)

    def dup(op, a128):
        return op(a128, jnp.roll(a128, _EMB, axis=-1))

    def round_(_, carry):
        t, c, v20, v21 = carry
        masked = jnp.where(x0 < t[:, None, :], x0, sent)
        m = dup(jnp.maximum, jnp.max(masked, axis=1))
        s = dup(jnp.add, jnp.sum(masked, axis=1))
        cnt = jnp.round(s * jnp.float32(-1e-35))
        v20 = jnp.where((c < 20.0) & (cnt >= 20.0), t, v20)
        v21 = jnp.where((c < 21.0) & (cnt >= 21.0), t, v21)
        return m, cnt, v20, v21

    zeros = jnp.zeros((_BB, 2 * _EMB), jnp.float32)
    inf = jnp.full((_BB, 2 * _EMB), jnp.inf, jnp.float32)
    _, _, v20, v21 = lax.fori_loop(
        0, 22, round_, (inf, zeros, zeros, zeros)
    )
    qs = v21 + jnp.float32(0.1) * (v20 - v21)
    s128 = jnp.sum(jnp.where(x0 >= qs[:, None, :], x0, 0.0), axis=1)
    pooled = dup(jnp.add, s128)[:, :_EMB] * jnp.float32(1.0 / _L)
    o_ref[...] = (
        jnp.dot(pooled, wt_ref[...], preferred_element_type=jnp.float32)
        + b_ref[...]
    )


def _tc_call(gathered3, wt, b2):
    grid = _BP // _BB
    return pl.pallas_call(
        _tc_body,
        grid=(grid,),
        in_specs=[
            pl.BlockSpec((_BB, _LH, 2 * _EMB), lambda i: (i, 0, 0)),
            pl.BlockSpec((_EMB, _OUT), lambda i: (0, 0)),
            pl.BlockSpec((1, _OUT), lambda i: (0, 0)),
        ],
        out_specs=pl.BlockSpec((_BB, _OUT), lambda i: (i, 0)),
        out_shape=jax.ShapeDtypeStruct((_BP, _OUT), jnp.float32),
    )(gathered3, wt, b2)


def kernel(tokens, table, W, b):
    tok2 = tokens.astype(jnp.int32).reshape(-1, _CH)
    wt = W.T
    b2 = b.reshape(1, _OUT)
    rows_per_slice = (_BP * _L) // _CH
    outs = []
    for p in range(_NPIPE):
        tok_p = lax.slice_in_dim(tok2, p * rows_per_slice,
                                 (p + 1) * rows_per_slice, axis=0)
        gathered = _sc_gather(tok_p, table)
        gathered3 = gathered.reshape(_BP, _LH, 2 * _EMB)
        outs.append(_tc_call(gathered3, wt, b2))
    return jnp.concatenate(outs, axis=0)


# speedup vs baseline: 1.0853x; 1.0853x over previous
import functools

import jax
import jax.numpy as jnp
from jax import lax
from jax.experimental import pallas as pl
from jax.experimental.pallas import tpu as pltpu
from jax.experimental.pallas import tpu_sc as plsc

_B = 4096
_L = 200
_EMB = 64
_OUT = 128

_NW = 32
_CH = 128
_NBUF = 5
_NPIPE = 4
_BP = _B // _NPIPE
_CHUNKS = (_BP * _L) // (_NW * _CH)
_NGROUP = _CHUNKS // _NBUF

_BB = 64
_LH = _L // 2
_SENT = -1e35


def _sc_gather(tok2, table):
    mesh = plsc.VectorSubcoreMesh(core_axis_name="c", subcore_axis_name="s")

    @functools.partial(
        pl.kernel,
        mesh=mesh,
        out_type=jax.ShapeDtypeStruct((_BP * _L, _EMB), jnp.float32),
        scratch_types=(
            [pltpu.VMEM((_CHUNKS, _CH), jnp.int32),
             pltpu.VMEM((_NBUF, _CH, _EMB), jnp.float32)]
            + [pltpu.SemaphoreType.DMA] * _NBUF
        ),
        compiler_params=pltpu.CompilerParams(use_tc_tiling_on_sc=False),
    )
    def k(tok_hbm, table_hbm, out_hbm, idx_v, rows_v, *sems):
        wid = lax.axis_index("s") * 2 + lax.axis_index("c")
        pltpu.sync_copy(tok_hbm.at[pl.ds(wid * _CHUNKS, _CHUNKS)], idx_v)
        for t in range(_NBUF):
            pltpu.async_copy(table_hbm.at[idx_v.at[t]], rows_v.at[t], sems[t])

        def body(g, carry):
            for t in range(_NBUF):
                j = g * _NBUF + t
                pltpu.make_async_copy(
                    table_hbm.at[idx_v.at[j]], rows_v.at[t], sems[t]
                ).wait()
                pltpu.sync_copy(
                    rows_v.at[t],
                    out_hbm.at[pl.ds((wid * _CHUNKS + j) * _CH, _CH)],
                )

                @pl.when(g < _NGROUP - 1)
                def _():
                    pltpu.async_copy(
                        table_hbm.at[idx_v.at[j + _NBUF]], rows_v.at[t], sems[t]
                    )

            return carry

        lax.fori_loop(0, _NGROUP, body, 0)

    return k(tok2, table)


def _tc_body(x_ref, wt_ref, b_ref, o_ref):
    x0 = x_ref[...]
    sent = jnp.float32(_SENT)

    def dup(op, a128):
        return op(a128, jnp.roll(a128, _EMB, axis=-1))

    def round_(_, carry):
        t, c, v20, v21 = carry
        v20 = jnp.where(c < 20.0, t, v20)
        v21 = jnp.where(c < 21.0, t, v21)
        masked = jnp.where(x0 < t[:, None, :], x0, sent)
        m = dup(jnp.maximum, jnp.max(masked, axis=1))
        s = dup(jnp.add, jnp.sum(masked, axis=1))
        cnt = jnp.round(s * jnp.float32(-1e-35))
        return m, cnt, v20, v21

    zeros = jnp.zeros((_BB, 2 * _EMB), jnp.float32)
    m0 = dup(jnp.maximum, jnp.max(x0, axis=1))
    _, _, v20, v21 = lax.fori_loop(
        0, 21, round_, (m0, zeros, zeros, zeros)
    )
    qs = v21 + jnp.float32(0.1) * (v20 - v21)
    s128 = jnp.sum(jnp.where(x0 >= qs[:, None, :], x0, 0.0), axis=1)
    pooled = dup(jnp.add, s128)[:, :_EMB] * jnp.float32(1.0 / _L)
    o_ref[...] = (
        jnp.dot(pooled, wt_ref[...], preferred_element_type=jnp.float32)
        + b_ref[...]
    )


def _tc_call(gathered3, wt, b2):
    grid = _BP // _BB
    return pl.pallas_call(
        _tc_body,
        grid=(grid,),
        in_specs=[
            pl.BlockSpec((_BB, _LH, 2 * _EMB), lambda i: (i, 0, 0)),
            pl.BlockSpec((_EMB, _OUT), lambda i: (0, 0)),
            pl.BlockSpec((1, _OUT), lambda i: (0, 0)),
        ],
        out_specs=pl.BlockSpec((_BB, _OUT), lambda i: (i, 0)),
        out_shape=jax.ShapeDtypeStruct((_BP, _OUT), jnp.float32),
    )(gathered3, wt, b2)


def kernel(tokens, table, W, b):
    tok2 = tokens.astype(jnp.int32).reshape(-1, _CH)
    wt = W.T
    b2 = b.reshape(1, _OUT)
    rows_per_slice = (_BP * _L) // _CH
    outs = []
    for p in range(_NPIPE):
        tok_p = lax.slice_in_dim(tok2, p * rows_per_slice,
                                 (p + 1) * rows_per_slice, axis=0)
        gathered = _sc_gather(tok_p, table)
        gathered3 = gathered.reshape(_BP, _LH, 2 * _EMB)
        outs.append(_tc_call(gathered3, wt, b2))
    return jnp.concatenate(outs, axis=0)
